# Initial kernel scaffold; baseline (speedup 1.0000x reference)
import jax
import jax.numpy as jnp
from jax.experimental import pallas as pl


def kernel(x, edge_index, rel_type, norm, weight, w_comp):
    raise NotImplementedError("write your pallas kernel here")


# speedup vs baseline: 1.1333x; 1.1333x over previous
import dataclasses
import functools

import jax
import jax.numpy as jnp
from jax import lax
from jax.experimental import pallas as pl
from jax.experimental.pallas import tpu as pltpu
from jax.experimental.pallas import tpu_sc as plsc

N = 10000
E = 320000
IN_FEAT = 128
OUT_FEAT = 128
NUM_RELS = 16
NUM_BASES = 8

NC = 2
NS = 16
NHALF = N // NC
EPW = 20480
E_PAD = NS * EPW
BLK = 128
SB = 4096
NSB = EPW // SB
SBROWS = SB // BLK
NBLK = SB // BLK
HROWS = 5040
ZROWS = 40


def _transform_body(x_ref, wc_ref, w16_ref, o_ref, xt_ref):
    r = pl.program_id(0)

    @pl.when(r == 0)
    def _():
        jj = lax.broadcasted_iota(jnp.int32, (IN_FEAT, NUM_RELS), 0)
        ss = lax.broadcasted_iota(jnp.int32, (IN_FEAT, NUM_RELS), 1)
        sel_s = (jj % NUM_RELS == ss).astype(jnp.float32)
        bb = lax.broadcasted_iota(jnp.int32, (NUM_BASES, 64), 0)
        kk = lax.broadcasted_iota(jnp.int32, (NUM_BASES, 64), 1)
        sel_b = (kk % NUM_BASES == bb).astype(jnp.float32)
        tiled = jnp.dot(sel_s, jnp.dot(wc_ref[...], sel_b),
                        preferred_element_type=jnp.float32)
        j2 = lax.broadcasted_iota(jnp.int32, (IN_FEAT, 64), 0)
        k2 = lax.broadcasted_iota(jnp.int32, (IN_FEAT, 64), 1)
        mask = (j2 // NUM_RELS == k2 // NUM_BASES).astype(jnp.float32)
        t_mat = tiled * mask
        xt_ref[...] = jnp.dot(x_ref[...], t_mat,
                              preferred_element_type=jnp.float32)

    o_ref[0] = jnp.dot(xt_ref[...], w16_ref[0],
                       preferred_element_type=jnp.float32)


def _transform(x, w_comp, w16):
    return pl.pallas_call(
        _transform_body,
        grid=(NUM_RELS,),
        in_specs=[
            pl.BlockSpec((N, IN_FEAT), lambda r: (0, 0)),
            pl.BlockSpec((NUM_RELS, NUM_BASES), lambda r: (0, 0)),
            pl.BlockSpec((1, 64, OUT_FEAT), lambda r: (r, 0, 0)),
        ],
        out_specs=pl.BlockSpec((1, N, OUT_FEAT), lambda r: (r, 0, 0)),
        out_shape=jax.ShapeDtypeStruct((NUM_RELS, N, OUT_FEAT), jnp.float32),
        scratch_shapes=[pltpu.VMEM((N, 64), jnp.float32)],
    )(x, w_comp, w16)


def _sc_body(t_hbm, src_hbm, rel_hbm, dst_hbm, norm_hbm, out_hbm,
             gidx, relb, didx, normb, rows, zbuf, hsh, sem):
    c = lax.axis_index("c")
    s = lax.axis_index("s")
    lo = c * NHALF

    @pl.loop(0, ZROWS)
    def _(i):
        for j in range(OUT_FEAT // 16):
            zbuf[i, pl.ds(j * 16, 16)] = jnp.zeros((16,), jnp.float32)

    @pl.loop(s, HROWS // ZROWS, step=NS)
    def _(b):
        pltpu.sync_copy(zbuf, hsh.at[pl.ds(b * ZROWS, ZROWS)])

    plsc.subcore_barrier()

    @pl.loop(0, NSB)
    def _(sb):
        base = s * EPW + sb * SB
        pltpu.sync_copy(src_hbm.at[s, pl.ds(sb * SBROWS, SBROWS)], gidx)
        pltpu.sync_copy(rel_hbm.at[s, pl.ds(sb * SBROWS, SBROWS)], relb)
        pltpu.sync_copy(dst_hbm.at[s, pl.ds(sb * SBROWS, SBROWS)], didx)
        pltpu.sync_copy(norm_hbm.at[pl.ds(base, SB)], normb)

        @pl.loop(0, SB // 16)
        def _(k):
            g, kk = k // (BLK // 16), k % (BLK // 16)
            sl = pl.ds(kk * 16, 16)
            gidx[g, sl] = relb[g, sl] * N + gidx[g, sl]
            dl = didx[g, sl] - lo
            ok = (dl >= 0) & (dl < NHALF)
            didx[g, sl] = jnp.where(ok, dl, NHALF)

        @pl.loop(0, NBLK)
        def _(g):
            pltpu.async_copy(t_hbm.at[gidx.at[g]], rows, sem).wait()

            @pl.loop(0, BLK)
            def _(i):
                nb = plsc.load_gather(
                    normb, [jnp.zeros((16,), jnp.int32) + (g * BLK + i)])
                for j in range(OUT_FEAT // 16):
                    sl = pl.ds(j * 16, 16)
                    rows[i, sl] = rows[i, sl] * nb

            pltpu.sync_copy(rows, hsh.at[didx.at[g]], add=True)

    plsc.subcore_barrier()

    @pl.loop(s, NHALF // ZROWS, step=NS)
    def _(b):
        pltpu.sync_copy(hsh.at[pl.ds(b * ZROWS, ZROWS)], zbuf)
        pltpu.sync_copy(zbuf, out_hbm.at[pl.ds(lo + b * ZROWS, ZROWS)])


def _sc_aggregate(t_flat, src3, rel3, dst3, norm_flat):
    mesh = plsc.VectorSubcoreMesh(core_axis_name="c", subcore_axis_name="s")
    cp = pltpu.CompilerParams()
    if "needs_layout_passes" in pltpu.CompilerParams.__dataclass_fields__:
        cp = dataclasses.replace(cp, needs_layout_passes=False)
    kern = pl.kernel(
        _sc_body,
        out_type=jax.ShapeDtypeStruct((N, OUT_FEAT), jnp.float32),
        mesh=mesh,
        scratch_types=[
            pltpu.VMEM((SBROWS, BLK), jnp.int32),
            pltpu.VMEM((SBROWS, BLK), jnp.int32),
            pltpu.VMEM((SBROWS, BLK), jnp.int32),
            pltpu.VMEM((SB,), jnp.float32),
            pltpu.VMEM((BLK, OUT_FEAT), jnp.float32),
            pltpu.VMEM((ZROWS, OUT_FEAT), jnp.float32),
            pltpu.VMEM_SHARED((HROWS, OUT_FEAT), jnp.float32),
            pltpu.SemaphoreType.DMA,
        ],
        compiler_params=cp,
    )
    return kern(t_flat, src3, rel3, dst3, norm_flat)


def kernel(x, edge_index, rel_type, norm, weight, w_comp):
    w16 = weight.reshape(NUM_RELS, 64, OUT_FEAT)
    t = _transform(x, w_comp, w16)
    t_flat = t.reshape(NUM_RELS * N, OUT_FEAT)

    pad = E_PAD - E
    src = jnp.concatenate([edge_index[0], jnp.zeros((pad,), jnp.int32)])
    dst = jnp.concatenate([edge_index[1], jnp.zeros((pad,), jnp.int32)])
    rel = jnp.concatenate([rel_type, jnp.zeros((pad,), jnp.int32)])
    nrm = jnp.concatenate([norm[:, 0], jnp.zeros((pad,), jnp.float32)])

    src3 = src.reshape(NS, EPW // BLK, BLK)
    rel3 = rel.reshape(NS, EPW // BLK, BLK)
    dst3 = dst.reshape(NS, EPW // BLK, BLK)

    return _sc_aggregate(t_flat, src3, rel3, dst3, nrm)


# speedup vs baseline: 1.3162x; 1.1614x over previous
import dataclasses
import functools

import jax
import jax.numpy as jnp
from jax import lax
from jax.experimental import pallas as pl
from jax.experimental.pallas import tpu as pltpu
from jax.experimental.pallas import tpu_sc as plsc

N = 10000
E = 320000
IN_FEAT = 128
OUT_FEAT = 128
NUM_RELS = 16
NUM_BASES = 8

NC = 2
NS = 16
NHALF = N // NC
EPW = 20480
E_PAD = NS * EPW
BLK = 128
SB = 4096
NSB = EPW // SB
SBROWS = SB // BLK
NBLK = SB // BLK
HROWS = 5040
ZROWS = 40


def _transform_body(x_ref, wc_ref, w16_ref, o_ref, xt_ref):
    r = pl.program_id(0)

    @pl.when(r == 0)
    def _():
        jj = lax.broadcasted_iota(jnp.int32, (IN_FEAT, NUM_RELS), 0)
        ss = lax.broadcasted_iota(jnp.int32, (IN_FEAT, NUM_RELS), 1)
        sel_s = (jj % NUM_RELS == ss).astype(jnp.float32)
        bb = lax.broadcasted_iota(jnp.int32, (NUM_BASES, 64), 0)
        kk = lax.broadcasted_iota(jnp.int32, (NUM_BASES, 64), 1)
        sel_b = (kk % NUM_BASES == bb).astype(jnp.float32)
        tiled = jnp.dot(sel_s, jnp.dot(wc_ref[...], sel_b),
                        preferred_element_type=jnp.float32)
        j2 = lax.broadcasted_iota(jnp.int32, (IN_FEAT, 64), 0)
        k2 = lax.broadcasted_iota(jnp.int32, (IN_FEAT, 64), 1)
        mask = (j2 // NUM_RELS == k2 // NUM_BASES).astype(jnp.float32)
        t_mat = tiled * mask
        xt_ref[...] = jnp.dot(x_ref[...], t_mat,
                              preferred_element_type=jnp.float32)

    o_ref[0] = jnp.dot(xt_ref[...], w16_ref[0],
                       preferred_element_type=jnp.float32)


def _transform(x, w_comp, w16):
    return pl.pallas_call(
        _transform_body,
        grid=(NUM_RELS,),
        in_specs=[
            pl.BlockSpec((N, IN_FEAT), lambda r: (0, 0)),
            pl.BlockSpec((NUM_RELS, NUM_BASES), lambda r: (0, 0)),
            pl.BlockSpec((1, 64, OUT_FEAT), lambda r: (r, 0, 0)),
        ],
        out_specs=pl.BlockSpec((1, N, OUT_FEAT), lambda r: (r, 0, 0)),
        out_shape=jax.ShapeDtypeStruct((NUM_RELS, N, OUT_FEAT), jnp.float32),
        scratch_shapes=[pltpu.VMEM((N, 64), jnp.float32)],
    )(x, w_comp, w16)


def _sc_body(t_hbm, src_hbm, rel_hbm, dst_hbm, norm_hbm, out_hbm,
             gidx, relb, didx, normb, rows, rows2, zbuf, hsh, sem, sem2):
    c = lax.axis_index("c")
    s = lax.axis_index("s")
    lo = c * NHALF

    @pl.loop(0, ZROWS)
    def _(i):
        for j in range(OUT_FEAT // 16):
            zbuf[i, pl.ds(j * 16, 16)] = jnp.zeros((16,), jnp.float32)

    @pl.loop(s, HROWS // ZROWS, step=NS)
    def _(b):
        pltpu.sync_copy(zbuf, hsh.at[pl.ds(b * ZROWS, ZROWS)])

    plsc.subcore_barrier()

    @pl.loop(0, NSB)
    def _(sb):
        base = s * EPW + sb * SB
        pltpu.sync_copy(src_hbm.at[s, pl.ds(sb * SBROWS, SBROWS)], gidx)
        pltpu.sync_copy(rel_hbm.at[s, pl.ds(sb * SBROWS, SBROWS)], relb)
        pltpu.sync_copy(dst_hbm.at[s, pl.ds(sb * SBROWS, SBROWS)], didx)
        pltpu.sync_copy(norm_hbm.at[pl.ds(base, SB)], normb)

        @pl.loop(0, SB // 16)
        def _(k):
            g, kk = k // (BLK // 16), k % (BLK // 16)
            sl = pl.ds(kk * 16, 16)
            gidx[g, sl] = relb[g, sl] * N + gidx[g, sl]
            dl = didx[g, sl] - lo
            ok = (dl >= 0) & (dl < NHALF)
            didx[g, sl] = jnp.where(ok, dl, NHALF)

        def gather_start(g, buf, sm):
            pltpu.make_async_copy(t_hbm.at[gidx.at[g]], buf, sm).start()

        def gather_wait(g, buf, sm):
            pltpu.make_async_copy(t_hbm.at[gidx.at[g]], buf, sm).wait()

        def scale_scatter(g, buf):
            @pl.loop(0, BLK)
            def _(i):
                nb = plsc.load_gather(
                    normb, [jnp.zeros((16,), jnp.int32) + (g * BLK + i)])
                for j in range(OUT_FEAT // 16):
                    sl = pl.ds(j * 16, 16)
                    buf[i, sl] = buf[i, sl] * nb

            pltpu.sync_copy(buf, hsh.at[didx.at[g]], add=True)

        gather_start(0, rows, sem)

        @pl.loop(0, NBLK, step=2)
        def _(g):
            gather_start(g + 1, rows2, sem2)
            gather_wait(g, rows, sem)
            scale_scatter(g, rows)

            @pl.when(g + 2 < NBLK)
            def _():
                gather_start(g + 2, rows, sem)

            gather_wait(g + 1, rows2, sem2)
            scale_scatter(g + 1, rows2)

    plsc.subcore_barrier()

    @pl.loop(s, NHALF // ZROWS, step=NS)
    def _(b):
        pltpu.sync_copy(hsh.at[pl.ds(b * ZROWS, ZROWS)], zbuf)
        pltpu.sync_copy(zbuf, out_hbm.at[pl.ds(lo + b * ZROWS, ZROWS)])


def _sc_aggregate(t_flat, src3, rel3, dst3, norm_flat):
    mesh = plsc.VectorSubcoreMesh(core_axis_name="c", subcore_axis_name="s")
    cp = pltpu.CompilerParams()
    if "needs_layout_passes" in pltpu.CompilerParams.__dataclass_fields__:
        cp = dataclasses.replace(cp, needs_layout_passes=False)
    kern = pl.kernel(
        _sc_body,
        out_type=jax.ShapeDtypeStruct((N, OUT_FEAT), jnp.float32),
        mesh=mesh,
        scratch_types=[
            pltpu.VMEM((SBROWS, BLK), jnp.int32),
            pltpu.VMEM((SBROWS, BLK), jnp.int32),
            pltpu.VMEM((SBROWS, BLK), jnp.int32),
            pltpu.VMEM((SB,), jnp.float32),
            pltpu.VMEM((BLK, OUT_FEAT), jnp.float32),
            pltpu.VMEM((BLK, OUT_FEAT), jnp.float32),
            pltpu.VMEM((ZROWS, OUT_FEAT), jnp.float32),
            pltpu.VMEM_SHARED((HROWS, OUT_FEAT), jnp.float32),
            pltpu.SemaphoreType.DMA,
            pltpu.SemaphoreType.DMA,
        ],
        compiler_params=cp,
    )
    return kern(t_flat, src3, rel3, dst3, norm_flat)


def kernel(x, edge_index, rel_type, norm, weight, w_comp):
    w16 = weight.reshape(NUM_RELS, 64, OUT_FEAT)
    t = _transform(x, w_comp, w16)
    t_flat = t.reshape(NUM_RELS * N, OUT_FEAT)

    pad = E_PAD - E
    src = jnp.concatenate([edge_index[0], jnp.zeros((pad,), jnp.int32)])
    dst = jnp.concatenate([edge_index[1], jnp.zeros((pad,), jnp.int32)])
    rel = jnp.concatenate([rel_type, jnp.zeros((pad,), jnp.int32)])
    nrm = jnp.concatenate([norm[:, 0], jnp.zeros((pad,), jnp.float32)])

    src3 = src.reshape(NS, EPW // BLK, BLK)
    rel3 = rel.reshape(NS, EPW // BLK, BLK)
    dst3 = dst.reshape(NS, EPW // BLK, BLK)

    return _sc_aggregate(t_flat, src3, rel3, dst3, nrm)
